# baseline (speedup 1.0000x reference)
import functools

import jax
import jax.numpy as jnp
from jax import lax
from jax.experimental import pallas as pl
from jax.experimental.pallas import tpu as pltpu
from jax.experimental.pallas import tpu_sc as plsc

VOCAB = 256
BATCH = 16384
D = 256

_info = plsc.get_sparse_core_info()
_NC, _NS, _L = _info.num_cores, _info.num_subcores, _info.num_lanes
_NW = _NC * _NS
_BPW = BATCH // _NW
_C = 128
_NCHUNK = _BPW // _C


@functools.partial(
    pl.kernel,
    mesh=plsc.VectorSubcoreMesh(core_axis_name="c", subcore_axis_name="s"),
    out_type=jax.ShapeDtypeStruct((BATCH, D), jnp.float32),
    scratch_types=[
        pltpu.VMEM((2 * _BPW,), jnp.int32),
        pltpu.VMEM((_NCHUNK, _C), jnp.int32),
        pltpu.VMEM((3, _C, D), jnp.float32),
        pltpu.SemaphoreType.DMA,
        pltpu.SemaphoreType.DMA,
        pltpu.SemaphoreType.DMA,
        pltpu.SemaphoreType.DMA,
        pltpu.SemaphoreType.DMA,
        pltpu.SemaphoreType.DMA,
    ],
)
def _lookup(x_hbm, table_hbm, out_hbm, x_v, idx_v, rows_v,
            g0, g1, g2, o0, o1, o2):
    wid = lax.axis_index("s") * _NC + lax.axis_index("c")
    base = wid * _BPW
    gsems = (g0, g1, g2)
    osems = (o0, o1, o2)
    pltpu.sync_copy(x_hbm.at[pl.ds(2 * base, 2 * _BPW)], x_v)

    lanes = lax.iota(jnp.int32, _L)
    lo_half = lanes < (_L // 2)
    perm_a = (lanes * 2) & (_L - 1)
    perm_b = (lanes * 2 + 1) & (_L - 1)

    def compute_idx(c):
        for i in range(_C // _L):
            g = c * (_C // _L) + i
            v0 = x_v[pl.ds(g * 2 * _L, _L)]
            v1 = x_v[pl.ds(g * 2 * _L + _L, _L)]
            a = jnp.where(lo_half, v0.at[perm_a].get(mode="promise_in_bounds"),
                          v1.at[perm_a].get(mode="promise_in_bounds"))
            b = jnp.where(lo_half, v0.at[perm_b].get(mode="promise_in_bounds"),
                          v1.at[perm_b].get(mode="promise_in_bounds"))
            idx_v[c, pl.ds(i * _L, _L)] = a * VOCAB + b

    def gather(c):
        return pltpu.async_copy(table_hbm.at[idx_v.at[c]], rows_v.at[c % 3],
                                gsems[c % 3])

    g = [None] * _NCHUNK
    o = [None] * _NCHUNK
    compute_idx(0)
    g[0] = gather(0)
    compute_idx(1)
    g[1] = gather(1)
    for c in range(2, _NCHUNK):
        compute_idx(c)
    for c in range(_NCHUNK):
        g[c].wait()
        o[c] = pltpu.async_copy(rows_v.at[c % 3],
                                out_hbm.at[pl.ds(base + c * _C, _C)],
                                osems[c % 3])
        if c + 2 < _NCHUNK:
            if c >= 1:
                o[c - 1].wait()
            g[c + 2] = gather(c + 2)
    o[_NCHUNK - 2].wait()
    o[_NCHUNK - 1].wait()


def kernel(x, lookup_table):
    return _lookup(x.reshape(-1), lookup_table)


# speedup vs baseline: 1.2992x; 1.2992x over previous
import functools

import jax
import jax.numpy as jnp
from jax import lax
from jax.experimental import pallas as pl
from jax.experimental.pallas import tpu as pltpu
from jax.experimental.pallas import tpu_sc as plsc

VOCAB = 256
BATCH = 16384
D = 256

_info = plsc.get_sparse_core_info()
_NC, _NS, _L = _info.num_cores, _info.num_subcores, _info.num_lanes
_NW = _NC * _NS
_BPW = BATCH // _NW
_C = 64
_NCHUNK = _BPW // _C
_NBUF = 6
_GDEPTH = 3


@functools.partial(
    pl.kernel,
    mesh=plsc.VectorSubcoreMesh(core_axis_name="c", subcore_axis_name="s"),
    out_type=jax.ShapeDtypeStruct((BATCH, D), jnp.float32),
    scratch_types=[
        pltpu.VMEM((_BPW,), jnp.int32),
        pltpu.VMEM((_BPW,), jnp.int32),
        pltpu.VMEM((_NCHUNK, _C), jnp.int32),
        pltpu.VMEM((_NBUF, _C, D), jnp.float32),
        pltpu.SemaphoreType.DMA,
        pltpu.SemaphoreType.DMA,
    ] + [pltpu.SemaphoreType.DMA] * (2 * _NBUF),
)
def _lookup(a_hbm, b_hbm, table_hbm, out_hbm, a_v, b_v, idx_v, rows_v,
            xsem_a, xsem_b, *sems):
    gsems = sems[:_NBUF]
    osems = sems[_NBUF:]
    wid = lax.axis_index("s") * _NC + lax.axis_index("c")
    base = wid * _BPW
    ca = pltpu.async_copy(a_hbm.at[pl.ds(base, _BPW)], a_v, xsem_a)
    cb = pltpu.async_copy(b_hbm.at[pl.ds(base, _BPW)], b_v, xsem_b)
    ca.wait()
    cb.wait()

    def compute_idx(c):
        for i in range(_C // _L):
            j = c * (_C // _L) + i
            va = a_v[pl.ds(j * _L, _L)]
            vb = b_v[pl.ds(j * _L, _L)]
            idx_v[c, pl.ds(i * _L, _L)] = va * VOCAB + vb

    def gather(c):
        return pltpu.async_copy(table_hbm.at[idx_v.at[c]], rows_v.at[c % _NBUF],
                                gsems[c % _NBUF])

    g = [None] * _NCHUNK
    o = [None] * _NCHUNK
    for c in range(_GDEPTH):
        compute_idx(c)
        g[c] = gather(c)
    for c in range(_GDEPTH, _NCHUNK):
        compute_idx(c)
    for c in range(_NCHUNK):
        g[c].wait()
        o[c] = pltpu.async_copy(rows_v.at[c % _NBUF],
                                out_hbm.at[pl.ds(base + c * _C, _C)],
                                osems[c % _NBUF])
        nxt = c + _GDEPTH
        if nxt < _NCHUNK:
            if nxt - _NBUF >= 0:
                o[nxt - _NBUF].wait()
            g[nxt] = gather(nxt)
    first_unwaited = max(0, _NCHUNK - _NBUF)
    for c in range(first_unwaited, _NCHUNK):
        o[c].wait()


def kernel(x, lookup_table):
    return _lookup(x[:, 0], x[:, 1], lookup_table)


# speedup vs baseline: 1.3576x; 1.0450x over previous
import functools

import jax
import jax.numpy as jnp
from jax import lax
from jax.experimental import pallas as pl
from jax.experimental.pallas import tpu as pltpu
from jax.experimental.pallas import tpu_sc as plsc

VOCAB = 256
BATCH = 16384
D = 256

_info = plsc.get_sparse_core_info()
_NC, _NS, _L = _info.num_cores, _info.num_subcores, _info.num_lanes
_NW = _NC * _NS
_BPW = BATCH // _NW
_C = 64
_NCHUNK = _BPW // _C
_NBUF = 7
_GDEPTH = 5


@functools.partial(
    pl.kernel,
    mesh=plsc.VectorSubcoreMesh(core_axis_name="c", subcore_axis_name="s"),
    out_type=jax.ShapeDtypeStruct((BATCH, D), jnp.float32),
    scratch_types=[
        pltpu.VMEM((_BPW,), jnp.int32),
        pltpu.VMEM((_BPW,), jnp.int32),
        pltpu.VMEM((_NCHUNK, _C), jnp.int32),
        pltpu.VMEM((_NBUF, _C, D), jnp.float32),
        pltpu.SemaphoreType.DMA,
        pltpu.SemaphoreType.DMA,
    ] + [pltpu.SemaphoreType.DMA] * (2 * _NBUF),
)
def _lookup(a_hbm, b_hbm, table_hbm, out_hbm, a_v, b_v, idx_v, rows_v,
            xsem_a, xsem_b, *sems):
    gsems = sems[:_NBUF]
    osems = sems[_NBUF:]
    wid = lax.axis_index("s") * _NC + lax.axis_index("c")
    base = wid * _BPW
    ca = pltpu.async_copy(a_hbm.at[pl.ds(base, _BPW)], a_v, xsem_a)
    cb = pltpu.async_copy(b_hbm.at[pl.ds(base, _BPW)], b_v, xsem_b)
    ca.wait()
    cb.wait()

    def compute_idx(c):
        for i in range(_C // _L):
            j = c * (_C // _L) + i
            va = a_v[pl.ds(j * _L, _L)]
            vb = b_v[pl.ds(j * _L, _L)]
            idx_v[c, pl.ds(i * _L, _L)] = va * VOCAB + vb

    def gather(c):
        return pltpu.async_copy(table_hbm.at[idx_v.at[c]], rows_v.at[c % _NBUF],
                                gsems[c % _NBUF])

    g = [None] * _NCHUNK
    o = [None] * _NCHUNK
    for c in range(_GDEPTH):
        compute_idx(c)
        g[c] = gather(c)
    for c in range(_GDEPTH, _NCHUNK):
        compute_idx(c)
    for c in range(_NCHUNK):
        g[c].wait()
        o[c] = pltpu.async_copy(rows_v.at[c % _NBUF],
                                out_hbm.at[pl.ds(base + c * _C, _C)],
                                osems[c % _NBUF])
        nxt = c + _GDEPTH
        if nxt < _NCHUNK:
            if nxt - _NBUF >= 0:
                o[nxt - _NBUF].wait()
            g[nxt] = gather(nxt)
    for c in range(max(0, _NCHUNK - _NBUF), _NCHUNK):
        o[c].wait()


def kernel(x, lookup_table):
    return _lookup(x[:, 0], x[:, 1], lookup_table)
